# baseline (speedup 1.0000x reference)
import functools

import jax
import jax.numpy as jnp
from jax import lax
from jax.experimental import pallas as pl
from jax.experimental.pallas import tpu as pltpu
from jax.experimental.pallas import tpu_sc as plsc

N = 50000
E = 80000
IN = 1024
FC = 128
HID = 37
EX = 20
EPS = 1e-5

L = 16
NCORES = 2
NSUB = 16
EPAD = 80128
Et = EPAD // NSUB
NV = Et // L
P = 5
C = 5120
NPAD = NCORES * P * C
CT = C // NSUB
CR = C // FC
CRP = 48
G = 128
NBUF = 2
MAXG = (Et + G - 1) // G
MAXC = MAXG * G
ZR = 16
NZ = CT // ZR

BA = 512
BN2 = 512


def _tc_pre_body(f_ref, wlT_ref, wrT_ref, bl_ref, z_ref, r_ref, exT_ref):
    xT = f_ref[:IN, :].astype(jnp.bfloat16)
    z_ref[...] = lax.dot_general(xT, wlT_ref[...], (((0,), (0,)), ((), ())),
                                 preferred_element_type=jnp.float32)
    r = lax.dot_general(xT, wrT_ref[...], (((0,), (0,)), ((), ())),
                        preferred_element_type=jnp.float32)
    r_ref[...] = r + bl_ref[...]
    exT_ref[...] = f_ref[IN:, :]


def _tc_pre(fT, wlT, wrT, bl2):
    nb = (N + BA - 1) // BA
    return pl.pallas_call(
        _tc_pre_body,
        grid=(nb,),
        in_specs=[
            pl.BlockSpec((1044, BA), lambda i: (0, i)),
            pl.BlockSpec((IN, FC), lambda i: (0, 0)),
            pl.BlockSpec((IN, FC), lambda i: (0, 0)),
            pl.BlockSpec((1, FC), lambda i: (0, 0)),
        ],
        out_specs=[
            pl.BlockSpec((BA, FC), lambda i: (i, 0)),
            pl.BlockSpec((BA, FC), lambda i: (i, 0)),
            pl.BlockSpec((EX, BA), lambda i: (0, i)),
        ],
        out_shape=[
            jax.ShapeDtypeStruct((N, FC), jnp.float32),
            jax.ShapeDtypeStruct((N, FC), jnp.float32),
            jax.ShapeDtypeStruct((EX, N), jnp.float32),
        ],
    )(fT, wlT, wrT, bl2)


def _sc_body(z_hbm, src_hbm, dst_hbm, zlc_hbm, agg_hbm, cnt_hbm,
             srcv, dstv, csrc, cdst, gb0, gb1, zbuf, lc, idxc, acc, accc,
             sg0, sg1, ss0, ss1, semz, semlc, semco):
    gbufs = (gb0, gb1)
    semg = (sg0, sg1)
    sems = (ss0, ss1)
    c = lax.axis_index("c")
    s = lax.axis_index("s")
    base_e = pl.multiple_of(s * Et, 8)
    pltpu.sync_copy(src_hbm.at[pl.ds(base_e, Et)], srcv)
    pltpu.sync_copy(dst_hbm.at[pl.ds(base_e, Et)], dstv)

    def _zb(rr, _):
        for j in range(FC // L):
            zbuf[rr, pl.ds(j * L, L)] = jnp.zeros((L,), jnp.float32)
        return 0
    lax.fori_loop(0, ZR, _zb, 0)
    for j in range(CRP // L):
        idxc[0, pl.ds(j * L, L)] = lax.iota(jnp.int32, L) + j * L

    def _wait_copyout():
        pltpu.make_async_copy(acc.at[pl.ds(0, CT)],
                              agg_hbm.at[pl.ds(0, CT)], semco).wait()

        @pl.when(s == 0)
        def _():
            pltpu.make_async_copy(accc.at[pl.ds(0, CR)],
                                  cnt_hbm.at[pl.ds(0, CR)], semco).wait()

    for p in range(P):
        lo = (c * P + p) * C

        if p > 0:
            _wait_copyout()

        for t in range(NZ):
            pltpu.async_copy(
                zbuf, acc.at[pl.ds(pl.multiple_of(s * CT + t * ZR, 8), ZR)], semz)

        @pl.when(s == 0)
        def _():
            pltpu.async_copy(zbuf.at[pl.ds(0, 8)], acc.at[pl.ds(C, 8)], semz)
            for t in range(CRP // ZR):
                pltpu.async_copy(zbuf, accc.at[pl.ds(t * ZR, ZR)], semz)

        for t in range(CRP // ZR):
            pltpu.async_copy(zlc_hbm, lc.at[pl.ds(t * ZR, ZR)], semlc)
        for t in range(CRP // ZR):
            pltpu.make_async_copy(zlc_hbm, lc.at[pl.ds(0, ZR)], semlc).wait()

        lov = jnp.full((L,), lo, jnp.int32)

        def _comp(i, cnt):
            off = pl.multiple_of(i * L, L)
            d = dstv[pl.ds(off, L)]
            sv = srcv[pl.ds(off, L)]
            dl = d - lov
            m = (dl >= jnp.zeros((L,), jnp.int32)) & (dl < jnp.full((L,), C, jnp.int32))
            mi = m.astype(jnp.int32)
            pos = jnp.full((L,), cnt, jnp.int32) + plsc.cumsum(mi) - mi
            plsc.store_scatter(csrc, [pos], sv, mask=m)
            dr = lax.shift_right_logical(dl, 7)
            dc = lax.bitwise_and(dl, jnp.full((L,), FC - 1, jnp.int32))
            plsc.store_scatter(
                cdst,
                [lax.shift_right_logical(pos, 7),
                 lax.bitwise_and(pos, jnp.full((L,), G - 1, jnp.int32))],
                dl, mask=m)
            occ, lastm = plsc.scan_count(dl, m)
            plsc.addupdate_scatter(lc, [dr, dc], occ.astype(jnp.float32),
                                   mask=lastm & m)
            return cnt + jnp.sum(mi)

        cnt = lax.fori_loop(0, NV, _comp, jnp.int32(0))
        ngroups = lax.shift_right_logical(cnt + (G - 1), 7)

        @pl.when(cnt > 0)
        def _():
            g0 = ngroups - 1
            for j in range(G // L):
                idx = jnp.full((L,), g0 * G + j * L, jnp.int32) + lax.iota(jnp.int32, L)
                m = idx >= jnp.full((L,), cnt, jnp.int32)
                plsc.store_scatter(csrc, [idx], jnp.zeros((L,), jnp.int32), mask=m)
                plsc.store_scatter(
                    cdst,
                    [jnp.full((L,), g0, jnp.int32),
                     lax.iota(jnp.int32, L) + j * L],
                    jnp.full((L,), C, jnp.int32), mask=m)

        for t in range(NZ):
            pltpu.make_async_copy(zbuf, acc.at[pl.ds(0, ZR)], semz).wait()

        @pl.when(s == 0)
        def _():
            pltpu.make_async_copy(zbuf.at[pl.ds(0, 8)], acc.at[pl.ds(C, 8)],
                                  semz).wait()
            for t in range(CRP // ZR):
                pltpu.make_async_copy(zbuf, accc.at[pl.ds(0, ZR)], semz).wait()

        plsc.subcore_barrier()

        nsteps = (ngroups + (NBUF - 1)) // NBUF

        def _super(ss_i, _):
            for b in range(NBUF):
                g = ss_i * NBUF + b

                @pl.when(g < ngroups)
                def _():
                    @pl.when(ss_i > 0)
                    def _():
                        pltpu.make_async_copy(gbufs[b], acc.at[pl.ds(0, G)],
                                              sems[b]).wait()
                    goff = pl.multiple_of(g * G, G)
                    pltpu.async_copy(z_hbm.at[csrc.at[pl.ds(goff, G)]],
                                     gbufs[b], semg[b])
            for b in range(NBUF):
                g = ss_i * NBUF + b

                @pl.when(g < ngroups)
                def _():
                    pltpu.make_async_copy(z_hbm.at[pl.ds(0, G)], gbufs[b],
                                          semg[b]).wait()
                    pltpu.async_copy(gbufs[b], acc.at[cdst.at[g]], sems[b],
                                     add=True)
            return 0

        lax.fori_loop(0, nsteps, _super, 0)
        for b in range(NBUF):
            @pl.when(ngroups > b)
            def _():
                pltpu.make_async_copy(gbufs[b], acc.at[pl.ds(0, G)],
                                      sems[b]).wait()

        pltpu.sync_copy(lc, accc.at[idxc.at[0]], add=True)

        plsc.subcore_barrier()

        row0 = s * CT
        pltpu.async_copy(acc.at[pl.ds(pl.multiple_of(row0, 8), CT)],
                         agg_hbm.at[pl.ds(pl.multiple_of(lo + row0, 8), CT)],
                         semco)

        @pl.when(s == 0)
        def _():
            pltpu.async_copy(accc.at[pl.ds(0, CR)],
                             cnt_hbm.at[pl.ds(pl.multiple_of((c * P + p) * CR, 8), CR)],
                             semco)

    _wait_copyout()


_sc_segsum = functools.partial(
    pl.kernel,
    out_type=(
        jax.ShapeDtypeStruct((NPAD, FC), jnp.float32),
        jax.ShapeDtypeStruct((NPAD // FC, FC), jnp.float32),
    ),
    mesh=plsc.VectorSubcoreMesh(core_axis_name="c", subcore_axis_name="s"),
    compiler_params=pltpu.CompilerParams(needs_layout_passes=False),
    scratch_types=[
        pltpu.VMEM((Et,), jnp.int32),
        pltpu.VMEM((Et,), jnp.int32),
        pltpu.VMEM((MAXC,), jnp.int32),
        pltpu.VMEM((MAXG, G), jnp.int32),
        pltpu.VMEM((G, FC), jnp.float32),
        pltpu.VMEM((G, FC), jnp.float32),
        pltpu.VMEM((ZR, FC), jnp.float32),
        pltpu.VMEM((CRP, FC), jnp.float32),
        pltpu.VMEM((8, CRP), jnp.int32),
        pltpu.VMEM_SHARED((C + 8, FC), jnp.float32),
        pltpu.VMEM_SHARED((CRP, FC), jnp.float32),
        pltpu.SemaphoreType.DMA,
        pltpu.SemaphoreType.DMA,
        pltpu.SemaphoreType.DMA,
        pltpu.SemaphoreType.DMA,
        pltpu.SemaphoreType.DMA,
        pltpu.SemaphoreType.DMA,
        pltpu.SemaphoreType.DMA,
    ],
)(_sc_body)


def _tc_post_body(agg_ref, cnt_ref, rsel_ref, lsel_ref, r_ref, exT_ref,
                  w1aT_ref, w1bT_ref, b1_ref,
                  gamma_ref, beta_ref, w2T_ref, b2_ref, out_ref):
    cb = cnt_ref[0]
    cr = lax.dot_general(rsel_ref[...], cb, (((1,), (0,)), ((), ())),
                         preferred_element_type=jnp.float32)
    cntcol = jnp.sum(cr * lsel_ref[...], axis=1, keepdims=True)
    mean = agg_ref[...] / jnp.maximum(cntcol, 1.0)
    h = jnp.maximum(mean + r_ref[...], 0.0)
    y = lax.dot_general(h, w1aT_ref[...], (((1,), (0,)), ((), ())),
                        preferred_element_type=jnp.float32)
    y = y + lax.dot_general(exT_ref[...], w1bT_ref[...], (((0,), (0,)), ((), ())),
                            preferred_element_type=jnp.float32)
    y = jnp.maximum(y + b1_ref[...], 0.0)
    scale = gamma_ref[...] * (1.0 / (1.0 + EPS) ** 0.5)
    y = y * scale + beta_ref[...]
    out_ref[...] = lax.dot_general(y, w2T_ref[...], (((1,), (0,)), ((), ())),
                                   preferred_element_type=jnp.float32) + b2_ref[...]


def _tc_post(agg, cnt3d, rsel, lsel, r, exT, w1aT, w1bT, b12, gamma2, beta2,
             w2T, b22):
    nb = (N + BN2 - 1) // BN2
    return pl.pallas_call(
        _tc_post_body,
        grid=(nb,),
        in_specs=[
            pl.BlockSpec((BN2, FC), lambda i: (i, 0)),
            pl.BlockSpec((1, BN2 // FC, FC), lambda i: (i, 0, 0)),
            pl.BlockSpec((BN2, BN2 // FC), lambda i: (0, 0)),
            pl.BlockSpec((BN2, FC), lambda i: (0, 0)),
            pl.BlockSpec((BN2, FC), lambda i: (i, 0)),
            pl.BlockSpec((EX, BN2), lambda i: (0, i)),
            pl.BlockSpec((FC, HID), lambda i: (0, 0)),
            pl.BlockSpec((EX, HID), lambda i: (0, 0)),
            pl.BlockSpec((1, HID), lambda i: (0, 0)),
            pl.BlockSpec((1, HID), lambda i: (0, 0)),
            pl.BlockSpec((1, HID), lambda i: (0, 0)),
            pl.BlockSpec((HID, 3), lambda i: (0, 0)),
            pl.BlockSpec((1, 3), lambda i: (0, 0)),
        ],
        out_specs=pl.BlockSpec((BN2, 3), lambda i: (i, 0)),
        out_shape=jax.ShapeDtypeStruct((N, 3), jnp.float32),
    )(agg, cnt3d, rsel, lsel, r, exT, w1aT, w1bT, b12, gamma2, beta2, w2T, b22)


def kernel(features, edges, edges2, edge_features, Wl, bl, Wr, W1, b1,
           gamma, beta, W2, b2):
    del edges2, edge_features
    src = edges[0]
    dst = edges[1]
    srcp = jnp.concatenate([src, jnp.zeros((EPAD - E,), jnp.int32)])
    dstp = jnp.concatenate([dst, jnp.full((EPAD - E,), NPAD, jnp.int32)])

    z, r, exT = _tc_pre(features.T, Wl.T.astype(jnp.bfloat16),
                        Wr.T.astype(jnp.bfloat16), bl.reshape(1, FC))
    agg, cnt2d = _sc_segsum(z, srcp, dstp, jnp.zeros((ZR, FC), jnp.float32))
    cnt3d = cnt2d.reshape(NPAD // BN2, BN2 // FC, FC)
    nidx = jnp.arange(BN2, dtype=jnp.int32)
    rsel = (nidx[:, None] // FC == jnp.arange(BN2 // FC, dtype=jnp.int32)[None, :]
            ).astype(jnp.float32)
    lsel = (nidx[:, None] % FC == jnp.arange(FC, dtype=jnp.int32)[None, :]
            ).astype(jnp.float32)
    out = _tc_post(agg, cnt3d, rsel, lsel, r, exT, W1[:, :FC].T, W1[:, FC:].T,
                   b1.reshape(1, HID), gamma.reshape(1, HID),
                   beta.reshape(1, HID), W2.T, b2.reshape(1, 3))
    return out


# speedup vs baseline: 1.0031x; 1.0031x over previous
import functools

import jax
import jax.numpy as jnp
from jax import lax
from jax.experimental import pallas as pl
from jax.experimental.pallas import tpu as pltpu
from jax.experimental.pallas import tpu_sc as plsc

N = 50000
E = 80000
IN = 1024
FC = 128
HID = 37
EX = 20
EPS = 1e-5

L = 16
NCORES = 2
NSUB = 16
EPAD = 80128
Et = EPAD // NSUB
NV = Et // L
P = 5
C = 5120
NPAD = NCORES * P * C
CT = C // NSUB
CR = C // FC
CRP = 48
G = 128
NBUF = 3
MAXG = (Et + G - 1) // G
MAXC = MAXG * G
ZR = 16
NZ = CT // ZR

BA = 512
BN2 = 512


def _tc_pre_body(f_ref, wlT_ref, wrT_ref, bl_ref, z_ref, r_ref, exT_ref):
    xT = f_ref[:IN, :].astype(jnp.bfloat16)
    z_ref[...] = lax.dot_general(xT, wlT_ref[...], (((0,), (0,)), ((), ())),
                                 preferred_element_type=jnp.float32)
    r = lax.dot_general(xT, wrT_ref[...], (((0,), (0,)), ((), ())),
                        preferred_element_type=jnp.float32)
    r_ref[...] = r + bl_ref[...]
    exT_ref[...] = f_ref[IN:, :]


def _tc_pre(fT, wlT, wrT, bl2):
    nb = (N + BA - 1) // BA
    return pl.pallas_call(
        _tc_pre_body,
        grid=(nb,),
        in_specs=[
            pl.BlockSpec((1044, BA), lambda i: (0, i)),
            pl.BlockSpec((IN, FC), lambda i: (0, 0)),
            pl.BlockSpec((IN, FC), lambda i: (0, 0)),
            pl.BlockSpec((1, FC), lambda i: (0, 0)),
        ],
        out_specs=[
            pl.BlockSpec((BA, FC), lambda i: (i, 0)),
            pl.BlockSpec((BA, FC), lambda i: (i, 0)),
            pl.BlockSpec((EX, BA), lambda i: (0, i)),
        ],
        out_shape=[
            jax.ShapeDtypeStruct((N, FC), jnp.float32),
            jax.ShapeDtypeStruct((N, FC), jnp.float32),
            jax.ShapeDtypeStruct((EX, N), jnp.float32),
        ],
    )(fT, wlT, wrT, bl2)


def _sc_body(z_hbm, src_hbm, dst_hbm, zlc_hbm, agg_hbm, cnt_hbm,
             srcv, dstv, csrc, cdst, gb0, gb1, gb2, zbuf, lc, idxc, acc, accc,
             sg0, sg1, sg2, ss0, ss1, ss2, semz, semlc, semco):
    gbufs = (gb0, gb1, gb2)
    semg = (sg0, sg1, sg2)
    sems = (ss0, ss1, ss2)
    c = lax.axis_index("c")
    s = lax.axis_index("s")
    base_e = pl.multiple_of(s * Et, 8)
    pltpu.sync_copy(src_hbm.at[pl.ds(base_e, Et)], srcv)
    pltpu.sync_copy(dst_hbm.at[pl.ds(base_e, Et)], dstv)

    def _zb(rr, _):
        for j in range(FC // L):
            zbuf[rr, pl.ds(j * L, L)] = jnp.zeros((L,), jnp.float32)
        return 0
    lax.fori_loop(0, ZR, _zb, 0)
    for j in range(CRP // L):
        idxc[0, pl.ds(j * L, L)] = lax.iota(jnp.int32, L) + j * L

    def _wait_copyout():
        pltpu.make_async_copy(acc.at[pl.ds(0, CT)],
                              agg_hbm.at[pl.ds(0, CT)], semco).wait()

        @pl.when(s == 0)
        def _():
            pltpu.make_async_copy(accc.at[pl.ds(0, CR)],
                                  cnt_hbm.at[pl.ds(0, CR)], semco).wait()

    for p in range(P):
        lo = (c * P + p) * C

        if p > 0:
            _wait_copyout()

        for t in range(NZ):
            pltpu.async_copy(
                zbuf, acc.at[pl.ds(pl.multiple_of(s * CT + t * ZR, 8), ZR)], semz)

        @pl.when(s == 0)
        def _():
            pltpu.async_copy(zbuf.at[pl.ds(0, 8)], acc.at[pl.ds(C, 8)], semz)
            for t in range(CRP // ZR):
                pltpu.async_copy(zbuf, accc.at[pl.ds(t * ZR, ZR)], semz)

        for t in range(CRP // ZR):
            pltpu.async_copy(zlc_hbm, lc.at[pl.ds(t * ZR, ZR)], semlc)
        for t in range(CRP // ZR):
            pltpu.make_async_copy(zlc_hbm, lc.at[pl.ds(0, ZR)], semlc).wait()

        lov = jnp.full((L,), lo, jnp.int32)

        def _comp(i, cnt):
            off = pl.multiple_of(i * L, L)
            d = dstv[pl.ds(off, L)]
            sv = srcv[pl.ds(off, L)]
            dl = d - lov
            m = (dl >= jnp.zeros((L,), jnp.int32)) & (dl < jnp.full((L,), C, jnp.int32))
            mi = m.astype(jnp.int32)
            pos = jnp.full((L,), cnt, jnp.int32) + plsc.cumsum(mi) - mi
            plsc.store_scatter(csrc, [pos], sv, mask=m)
            dr = lax.shift_right_logical(dl, 7)
            dc = lax.bitwise_and(dl, jnp.full((L,), FC - 1, jnp.int32))
            plsc.store_scatter(
                cdst,
                [lax.shift_right_logical(pos, 7),
                 lax.bitwise_and(pos, jnp.full((L,), G - 1, jnp.int32))],
                dl, mask=m)
            occ, lastm = plsc.scan_count(dl, m)
            plsc.addupdate_scatter(lc, [dr, dc], occ.astype(jnp.float32),
                                   mask=lastm & m)
            return cnt + jnp.sum(mi)

        cnt = lax.fori_loop(0, NV, _comp, jnp.int32(0))
        ngroups = lax.shift_right_logical(cnt + (G - 1), 7)

        @pl.when(cnt > 0)
        def _():
            g0 = ngroups - 1
            for j in range(G // L):
                idx = jnp.full((L,), g0 * G + j * L, jnp.int32) + lax.iota(jnp.int32, L)
                m = idx >= jnp.full((L,), cnt, jnp.int32)
                plsc.store_scatter(csrc, [idx], jnp.zeros((L,), jnp.int32), mask=m)
                plsc.store_scatter(
                    cdst,
                    [jnp.full((L,), g0, jnp.int32),
                     lax.iota(jnp.int32, L) + j * L],
                    jnp.full((L,), C, jnp.int32), mask=m)

        for t in range(NZ):
            pltpu.make_async_copy(zbuf, acc.at[pl.ds(0, ZR)], semz).wait()

        @pl.when(s == 0)
        def _():
            pltpu.make_async_copy(zbuf.at[pl.ds(0, 8)], acc.at[pl.ds(C, 8)],
                                  semz).wait()
            for t in range(CRP // ZR):
                pltpu.make_async_copy(zbuf, accc.at[pl.ds(0, ZR)], semz).wait()

        plsc.subcore_barrier()

        nsteps = (ngroups + (NBUF - 1)) // NBUF

        def _super(ss_i, _):
            for b in range(NBUF):
                g = ss_i * NBUF + b

                @pl.when(g < ngroups)
                def _():
                    @pl.when(ss_i > 0)
                    def _():
                        pltpu.make_async_copy(gbufs[b], acc.at[pl.ds(0, G)],
                                              sems[b]).wait()
                    goff = pl.multiple_of(g * G, G)
                    pltpu.async_copy(z_hbm.at[csrc.at[pl.ds(goff, G)]],
                                     gbufs[b], semg[b])
            for b in range(NBUF):
                g = ss_i * NBUF + b

                @pl.when(g < ngroups)
                def _():
                    pltpu.make_async_copy(z_hbm.at[pl.ds(0, G)], gbufs[b],
                                          semg[b]).wait()
                    pltpu.async_copy(gbufs[b], acc.at[cdst.at[g]], sems[b],
                                     add=True)
            return 0

        lax.fori_loop(0, nsteps, _super, 0)
        for b in range(NBUF):
            @pl.when(ngroups > b)
            def _():
                pltpu.make_async_copy(gbufs[b], acc.at[pl.ds(0, G)],
                                      sems[b]).wait()

        pltpu.sync_copy(lc, accc.at[idxc.at[0]], add=True)

        plsc.subcore_barrier()

        row0 = s * CT
        pltpu.async_copy(acc.at[pl.ds(pl.multiple_of(row0, 8), CT)],
                         agg_hbm.at[pl.ds(pl.multiple_of(lo + row0, 8), CT)],
                         semco)

        @pl.when(s == 0)
        def _():
            pltpu.async_copy(accc.at[pl.ds(0, CR)],
                             cnt_hbm.at[pl.ds(pl.multiple_of((c * P + p) * CR, 8), CR)],
                             semco)

    _wait_copyout()


_sc_segsum = functools.partial(
    pl.kernel,
    out_type=(
        jax.ShapeDtypeStruct((NPAD, FC), jnp.float32),
        jax.ShapeDtypeStruct((NPAD // FC, FC), jnp.float32),
    ),
    mesh=plsc.VectorSubcoreMesh(core_axis_name="c", subcore_axis_name="s"),
    compiler_params=pltpu.CompilerParams(needs_layout_passes=False),
    scratch_types=[
        pltpu.VMEM((Et,), jnp.int32),
        pltpu.VMEM((Et,), jnp.int32),
        pltpu.VMEM((MAXC,), jnp.int32),
        pltpu.VMEM((MAXG, G), jnp.int32),
        pltpu.VMEM((G, FC), jnp.float32),
        pltpu.VMEM((G, FC), jnp.float32),
        pltpu.VMEM((G, FC), jnp.float32),
        pltpu.VMEM((ZR, FC), jnp.float32),
        pltpu.VMEM((CRP, FC), jnp.float32),
        pltpu.VMEM((8, CRP), jnp.int32),
        pltpu.VMEM_SHARED((C + 8, FC), jnp.float32),
        pltpu.VMEM_SHARED((CRP, FC), jnp.float32),
        pltpu.SemaphoreType.DMA,
        pltpu.SemaphoreType.DMA,
        pltpu.SemaphoreType.DMA,
        pltpu.SemaphoreType.DMA,
        pltpu.SemaphoreType.DMA,
        pltpu.SemaphoreType.DMA,
        pltpu.SemaphoreType.DMA,
        pltpu.SemaphoreType.DMA,
        pltpu.SemaphoreType.DMA,
    ],
)(_sc_body)


def _tc_post_body(agg_ref, cnt_ref, rsel_ref, lsel_ref, r_ref, exT_ref,
                  w1aT_ref, w1bT_ref, b1_ref,
                  gamma_ref, beta_ref, w2T_ref, b2_ref, out_ref):
    cb = cnt_ref[0]
    cr = lax.dot_general(rsel_ref[...], cb, (((1,), (0,)), ((), ())),
                         preferred_element_type=jnp.float32)
    cntcol = jnp.sum(cr * lsel_ref[...], axis=1, keepdims=True)
    mean = agg_ref[...] / jnp.maximum(cntcol, 1.0)
    h = jnp.maximum(mean + r_ref[...], 0.0)
    y = lax.dot_general(h, w1aT_ref[...], (((1,), (0,)), ((), ())),
                        preferred_element_type=jnp.float32)
    y = y + lax.dot_general(exT_ref[...], w1bT_ref[...], (((0,), (0,)), ((), ())),
                            preferred_element_type=jnp.float32)
    y = jnp.maximum(y + b1_ref[...], 0.0)
    scale = gamma_ref[...] * (1.0 / (1.0 + EPS) ** 0.5)
    y = y * scale + beta_ref[...]
    out_ref[...] = lax.dot_general(y, w2T_ref[...], (((1,), (0,)), ((), ())),
                                   preferred_element_type=jnp.float32) + b2_ref[...]


def _tc_post(agg, cnt3d, rsel, lsel, r, exT, w1aT, w1bT, b12, gamma2, beta2,
             w2T, b22):
    nb = (N + BN2 - 1) // BN2
    return pl.pallas_call(
        _tc_post_body,
        grid=(nb,),
        in_specs=[
            pl.BlockSpec((BN2, FC), lambda i: (i, 0)),
            pl.BlockSpec((1, BN2 // FC, FC), lambda i: (i, 0, 0)),
            pl.BlockSpec((BN2, BN2 // FC), lambda i: (0, 0)),
            pl.BlockSpec((BN2, FC), lambda i: (0, 0)),
            pl.BlockSpec((BN2, FC), lambda i: (i, 0)),
            pl.BlockSpec((EX, BN2), lambda i: (0, i)),
            pl.BlockSpec((FC, HID), lambda i: (0, 0)),
            pl.BlockSpec((EX, HID), lambda i: (0, 0)),
            pl.BlockSpec((1, HID), lambda i: (0, 0)),
            pl.BlockSpec((1, HID), lambda i: (0, 0)),
            pl.BlockSpec((1, HID), lambda i: (0, 0)),
            pl.BlockSpec((HID, 3), lambda i: (0, 0)),
            pl.BlockSpec((1, 3), lambda i: (0, 0)),
        ],
        out_specs=pl.BlockSpec((BN2, 3), lambda i: (i, 0)),
        out_shape=jax.ShapeDtypeStruct((N, 3), jnp.float32),
    )(agg, cnt3d, rsel, lsel, r, exT, w1aT, w1bT, b12, gamma2, beta2, w2T, b22)


def kernel(features, edges, edges2, edge_features, Wl, bl, Wr, W1, b1,
           gamma, beta, W2, b2):
    del edges2, edge_features
    src = edges[0]
    dst = edges[1]
    srcp = jnp.concatenate([src, jnp.zeros((EPAD - E,), jnp.int32)])
    dstp = jnp.concatenate([dst, jnp.full((EPAD - E,), NPAD, jnp.int32)])

    z, r, exT = _tc_pre(features.T, Wl.T.astype(jnp.bfloat16),
                        Wr.T.astype(jnp.bfloat16), bl.reshape(1, FC))
    agg, cnt2d = _sc_segsum(z, srcp, dstp, jnp.zeros((ZR, FC), jnp.float32))
    cnt3d = cnt2d.reshape(NPAD // BN2, BN2 // FC, FC)
    nidx = jnp.arange(BN2, dtype=jnp.int32)
    rsel = (nidx[:, None] // FC == jnp.arange(BN2 // FC, dtype=jnp.int32)[None, :]
            ).astype(jnp.float32)
    lsel = (nidx[:, None] % FC == jnp.arange(FC, dtype=jnp.int32)[None, :]
            ).astype(jnp.float32)
    out = _tc_post(agg, cnt3d, rsel, lsel, r, exT, W1[:, :FC].T, W1[:, FC:].T,
                   b1.reshape(1, HID), gamma.reshape(1, HID),
                   beta.reshape(1, HID), W2.T, b2.reshape(1, 3))
    return out


# speedup vs baseline: 1.1033x; 1.0999x over previous
import functools

import jax
import jax.numpy as jnp
from jax import lax
from jax.experimental import pallas as pl
from jax.experimental.pallas import tpu as pltpu
from jax.experimental.pallas import tpu_sc as plsc

N = 50000
E = 80000
IN = 1024
FC = 128
HID = 37
EX = 20
EPS = 1e-5

L = 16
NCORES = 2
NSUB = 16
EPAD = 80128
Et = EPAD // NSUB
NV = Et // L
P = 5
C = 5120
NPAD = NCORES * P * C
CT = C // NSUB
CR = C // FC
CRP = 48
G = 128
NBUF = 3
MAXG = (Et + G - 1) // G
MAXC = MAXG * G
ZR = 16
NZ = CT // ZR

BA = 1024
BN2 = 1024


def _tc_pre_body(f_ref, wlT_ref, wrT_ref, bl_ref, z_ref, r_ref, exT_ref):
    xT = f_ref[:IN, :].astype(jnp.bfloat16)
    z_ref[...] = lax.dot_general(xT, wlT_ref[...], (((0,), (0,)), ((), ())),
                                 preferred_element_type=jnp.float32)
    r = lax.dot_general(xT, wrT_ref[...], (((0,), (0,)), ((), ())),
                        preferred_element_type=jnp.float32)
    r_ref[...] = r + bl_ref[...]
    exT_ref[...] = f_ref[IN:, :]


def _tc_pre(fT, wlT, wrT, bl2):
    nb = (N + BA - 1) // BA
    return pl.pallas_call(
        _tc_pre_body,
        grid=(nb,),
        in_specs=[
            pl.BlockSpec((1044, BA), lambda i: (0, i)),
            pl.BlockSpec((IN, FC), lambda i: (0, 0)),
            pl.BlockSpec((IN, FC), lambda i: (0, 0)),
            pl.BlockSpec((1, FC), lambda i: (0, 0)),
        ],
        out_specs=[
            pl.BlockSpec((BA, FC), lambda i: (i, 0)),
            pl.BlockSpec((BA, FC), lambda i: (i, 0)),
            pl.BlockSpec((EX, BA), lambda i: (0, i)),
        ],
        out_shape=[
            jax.ShapeDtypeStruct((N, FC), jnp.float32),
            jax.ShapeDtypeStruct((N, FC), jnp.float32),
            jax.ShapeDtypeStruct((EX, N), jnp.float32),
        ],
    )(fT, wlT, wrT, bl2)


def _sc_body(z_hbm, src_hbm, dst_hbm, zlc_hbm, agg_hbm, cnt_hbm,
             srcv, dstv, csrc, cdst, gb0, gb1, gb2, zbuf, lc, idxc, acc, accc,
             sg0, sg1, sg2, ss0, ss1, ss2, semz, semlc, semco):
    gbufs = (gb0, gb1, gb2)
    semg = (sg0, sg1, sg2)
    sems = (ss0, ss1, ss2)
    c = lax.axis_index("c")
    s = lax.axis_index("s")
    base_e = pl.multiple_of(s * Et, 8)
    pltpu.sync_copy(src_hbm.at[pl.ds(base_e, Et)], srcv)
    pltpu.sync_copy(dst_hbm.at[pl.ds(base_e, Et)], dstv)

    def _zb(rr, _):
        for j in range(FC // L):
            zbuf[rr, pl.ds(j * L, L)] = jnp.zeros((L,), jnp.float32)
        return 0
    lax.fori_loop(0, ZR, _zb, 0)
    for j in range(CRP // L):
        idxc[0, pl.ds(j * L, L)] = lax.iota(jnp.int32, L) + j * L

    def _wait_copyout():
        pltpu.make_async_copy(acc.at[pl.ds(0, CT)],
                              agg_hbm.at[pl.ds(0, CT)], semco).wait()

        @pl.when(s == 0)
        def _():
            pltpu.make_async_copy(accc.at[pl.ds(0, CR)],
                                  cnt_hbm.at[pl.ds(0, CR)], semco).wait()

    for p in range(P):
        lo = (c * P + p) * C

        if p > 0:
            _wait_copyout()

        for t in range(NZ):
            pltpu.async_copy(
                zbuf, acc.at[pl.ds(pl.multiple_of(s * CT + t * ZR, 8), ZR)], semz)

        @pl.when(s == 0)
        def _():
            pltpu.async_copy(zbuf.at[pl.ds(0, 8)], acc.at[pl.ds(C, 8)], semz)
            for t in range(CRP // ZR):
                pltpu.async_copy(zbuf, accc.at[pl.ds(t * ZR, ZR)], semz)

        for t in range(CRP // ZR):
            pltpu.async_copy(zlc_hbm, lc.at[pl.ds(t * ZR, ZR)], semlc)
        for t in range(CRP // ZR):
            pltpu.make_async_copy(zlc_hbm, lc.at[pl.ds(0, ZR)], semlc).wait()

        lov = jnp.full((L,), lo, jnp.int32)

        def _comp(i, cnt):
            off = pl.multiple_of(i * L, L)
            d = dstv[pl.ds(off, L)]
            sv = srcv[pl.ds(off, L)]
            dl = d - lov
            m = (dl >= jnp.zeros((L,), jnp.int32)) & (dl < jnp.full((L,), C, jnp.int32))
            mi = m.astype(jnp.int32)
            pos = jnp.full((L,), cnt, jnp.int32) + plsc.cumsum(mi) - mi
            plsc.store_scatter(csrc, [pos], sv, mask=m)
            dr = lax.shift_right_logical(dl, 7)
            dc = lax.bitwise_and(dl, jnp.full((L,), FC - 1, jnp.int32))
            plsc.store_scatter(
                cdst,
                [lax.shift_right_logical(pos, 7),
                 lax.bitwise_and(pos, jnp.full((L,), G - 1, jnp.int32))],
                dl, mask=m)
            occ, lastm = plsc.scan_count(dl, m)
            plsc.addupdate_scatter(lc, [dr, dc], occ.astype(jnp.float32),
                                   mask=lastm & m)
            return cnt + jnp.sum(mi)

        cnt = lax.fori_loop(0, NV, _comp, jnp.int32(0))
        ngroups = lax.shift_right_logical(cnt + (G - 1), 7)

        @pl.when(cnt > 0)
        def _():
            g0 = ngroups - 1
            for j in range(G // L):
                idx = jnp.full((L,), g0 * G + j * L, jnp.int32) + lax.iota(jnp.int32, L)
                m = idx >= jnp.full((L,), cnt, jnp.int32)
                plsc.store_scatter(csrc, [idx], jnp.zeros((L,), jnp.int32), mask=m)
                plsc.store_scatter(
                    cdst,
                    [jnp.full((L,), g0, jnp.int32),
                     lax.iota(jnp.int32, L) + j * L],
                    jnp.full((L,), C, jnp.int32), mask=m)

        for t in range(NZ):
            pltpu.make_async_copy(zbuf, acc.at[pl.ds(0, ZR)], semz).wait()

        @pl.when(s == 0)
        def _():
            pltpu.make_async_copy(zbuf.at[pl.ds(0, 8)], acc.at[pl.ds(C, 8)],
                                  semz).wait()
            for t in range(CRP // ZR):
                pltpu.make_async_copy(zbuf, accc.at[pl.ds(0, ZR)], semz).wait()

        plsc.subcore_barrier()

        nsteps = (ngroups + (NBUF - 1)) // NBUF

        def _super(ss_i, _):
            for b in range(NBUF):
                g = ss_i * NBUF + b

                @pl.when(g < ngroups)
                def _():
                    @pl.when(ss_i > 0)
                    def _():
                        pltpu.make_async_copy(gbufs[b], acc.at[pl.ds(0, G)],
                                              sems[b]).wait()
                    goff = pl.multiple_of(g * G, G)
                    pltpu.async_copy(z_hbm.at[csrc.at[pl.ds(goff, G)]],
                                     gbufs[b], semg[b])
            for b in range(NBUF):
                g = ss_i * NBUF + b

                @pl.when(g < ngroups)
                def _():
                    pltpu.make_async_copy(z_hbm.at[pl.ds(0, G)], gbufs[b],
                                          semg[b]).wait()
                    pltpu.async_copy(gbufs[b], acc.at[cdst.at[g]], sems[b],
                                     add=True)
            return 0

        lax.fori_loop(0, nsteps, _super, 0)
        for b in range(NBUF):
            @pl.when(ngroups > b)
            def _():
                pltpu.make_async_copy(gbufs[b], acc.at[pl.ds(0, G)],
                                      sems[b]).wait()

        pltpu.sync_copy(lc, accc.at[idxc.at[0]], add=True)

        plsc.subcore_barrier()

        row0 = s * CT
        pltpu.async_copy(acc.at[pl.ds(pl.multiple_of(row0, 8), CT)],
                         agg_hbm.at[pl.ds(pl.multiple_of(lo + row0, 8), CT)],
                         semco)

        @pl.when(s == 0)
        def _():
            pltpu.async_copy(accc.at[pl.ds(0, CR)],
                             cnt_hbm.at[pl.ds(pl.multiple_of((c * P + p) * CR, 8), CR)],
                             semco)

    _wait_copyout()


_sc_segsum = functools.partial(
    pl.kernel,
    out_type=(
        jax.ShapeDtypeStruct((NPAD, FC), jnp.float32),
        jax.ShapeDtypeStruct((NPAD // FC, FC), jnp.float32),
    ),
    mesh=plsc.VectorSubcoreMesh(core_axis_name="c", subcore_axis_name="s"),
    compiler_params=pltpu.CompilerParams(needs_layout_passes=False),
    scratch_types=[
        pltpu.VMEM((Et,), jnp.int32),
        pltpu.VMEM((Et,), jnp.int32),
        pltpu.VMEM((MAXC,), jnp.int32),
        pltpu.VMEM((MAXG, G), jnp.int32),
        pltpu.VMEM((G, FC), jnp.float32),
        pltpu.VMEM((G, FC), jnp.float32),
        pltpu.VMEM((G, FC), jnp.float32),
        pltpu.VMEM((ZR, FC), jnp.float32),
        pltpu.VMEM((CRP, FC), jnp.float32),
        pltpu.VMEM((8, CRP), jnp.int32),
        pltpu.VMEM_SHARED((C + 8, FC), jnp.float32),
        pltpu.VMEM_SHARED((CRP, FC), jnp.float32),
        pltpu.SemaphoreType.DMA,
        pltpu.SemaphoreType.DMA,
        pltpu.SemaphoreType.DMA,
        pltpu.SemaphoreType.DMA,
        pltpu.SemaphoreType.DMA,
        pltpu.SemaphoreType.DMA,
        pltpu.SemaphoreType.DMA,
        pltpu.SemaphoreType.DMA,
        pltpu.SemaphoreType.DMA,
    ],
)(_sc_body)


def _tc_post_body(agg_ref, cnt_ref, rsel_ref, lsel_ref, r_ref, exT_ref,
                  w1aT_ref, w1bT_ref, b1_ref,
                  gamma_ref, beta_ref, w2T_ref, b2_ref, out_ref):
    cb = cnt_ref[0]
    cr = lax.dot_general(rsel_ref[...], cb, (((1,), (0,)), ((), ())),
                         preferred_element_type=jnp.float32)
    cntcol = jnp.sum(cr * lsel_ref[...], axis=1, keepdims=True)
    mean = agg_ref[...] / jnp.maximum(cntcol, 1.0)
    h = jnp.maximum(mean + r_ref[...], 0.0)
    y = lax.dot_general(h, w1aT_ref[...], (((1,), (0,)), ((), ())),
                        preferred_element_type=jnp.float32)
    y = y + lax.dot_general(exT_ref[...], w1bT_ref[...], (((0,), (0,)), ((), ())),
                            preferred_element_type=jnp.float32)
    y = jnp.maximum(y + b1_ref[...], 0.0)
    scale = gamma_ref[...] * (1.0 / (1.0 + EPS) ** 0.5)
    y = y * scale + beta_ref[...]
    out_ref[...] = lax.dot_general(y, w2T_ref[...], (((1,), (0,)), ((), ())),
                                   preferred_element_type=jnp.float32) + b2_ref[...]


def _tc_post(agg, cnt3d, rsel, lsel, r, exT, w1aT, w1bT, b12, gamma2, beta2,
             w2T, b22):
    nb = (N + BN2 - 1) // BN2
    return pl.pallas_call(
        _tc_post_body,
        grid=(nb,),
        in_specs=[
            pl.BlockSpec((BN2, FC), lambda i: (i, 0)),
            pl.BlockSpec((1, BN2 // FC, FC), lambda i: (i, 0, 0)),
            pl.BlockSpec((BN2, BN2 // FC), lambda i: (0, 0)),
            pl.BlockSpec((BN2, FC), lambda i: (0, 0)),
            pl.BlockSpec((BN2, FC), lambda i: (i, 0)),
            pl.BlockSpec((EX, BN2), lambda i: (0, i)),
            pl.BlockSpec((FC, HID), lambda i: (0, 0)),
            pl.BlockSpec((EX, HID), lambda i: (0, 0)),
            pl.BlockSpec((1, HID), lambda i: (0, 0)),
            pl.BlockSpec((1, HID), lambda i: (0, 0)),
            pl.BlockSpec((1, HID), lambda i: (0, 0)),
            pl.BlockSpec((HID, 3), lambda i: (0, 0)),
            pl.BlockSpec((1, 3), lambda i: (0, 0)),
        ],
        out_specs=pl.BlockSpec((BN2, 3), lambda i: (i, 0)),
        out_shape=jax.ShapeDtypeStruct((N, 3), jnp.float32),
    )(agg, cnt3d, rsel, lsel, r, exT, w1aT, w1bT, b12, gamma2, beta2, w2T, b22)


def kernel(features, edges, edges2, edge_features, Wl, bl, Wr, W1, b1,
           gamma, beta, W2, b2):
    del edges2, edge_features
    src = edges[0]
    dst = edges[1]
    srcp = jnp.concatenate([src, jnp.zeros((EPAD - E,), jnp.int32)])
    dstp = jnp.concatenate([dst, jnp.full((EPAD - E,), NPAD, jnp.int32)])

    z, r, exT = _tc_pre(features.T, Wl.T.astype(jnp.bfloat16),
                        Wr.T.astype(jnp.bfloat16), bl.reshape(1, FC))
    agg, cnt2d = _sc_segsum(z, srcp, dstp, jnp.zeros((ZR, FC), jnp.float32))
    cnt3d = cnt2d.reshape(NPAD // BN2, BN2 // FC, FC)
    nidx = jnp.arange(BN2, dtype=jnp.int32)
    rsel = (nidx[:, None] // FC == jnp.arange(BN2 // FC, dtype=jnp.int32)[None, :]
            ).astype(jnp.float32)
    lsel = (nidx[:, None] % FC == jnp.arange(FC, dtype=jnp.int32)[None, :]
            ).astype(jnp.float32)
    out = _tc_post(agg, cnt3d, rsel, lsel, r, exT, W1[:, :FC].T, W1[:, FC:].T,
                   b1.reshape(1, HID), gamma.reshape(1, HID),
                   beta.reshape(1, HID), W2.T, b2.reshape(1, 3))
    return out


# speedup vs baseline: 1.1280x; 1.0224x over previous
import functools

import jax
import jax.numpy as jnp
from jax import lax
from jax.experimental import pallas as pl
from jax.experimental.pallas import tpu as pltpu
from jax.experimental.pallas import tpu_sc as plsc

N = 50000
E = 80000
IN = 1024
FC = 128
HID = 37
EX = 20
EPS = 1e-5

L = 16
NCORES = 2
NSUB = 16
EPAD = 80128
Et = EPAD // NSUB
NV = Et // L
P = 5
C = 5120
NPAD = NCORES * P * C
CT = C // NSUB
CR = C // FC
CRP = 48
G = 128
NBUF = 3
MAXG = (Et + G - 1) // G
MAXC = MAXG * G
ZR = 16
NZ = CT // ZR

BA = 2048
BN2 = 1024


def _tc_pre_body(f_ref, wlT_ref, wrT_ref, bl_ref, z_ref, r_ref, exT_ref):
    xT = f_ref[:IN, :].astype(jnp.bfloat16)
    z_ref[...] = lax.dot_general(xT, wlT_ref[...], (((0,), (0,)), ((), ())),
                                 preferred_element_type=jnp.float32)
    r = lax.dot_general(xT, wrT_ref[...], (((0,), (0,)), ((), ())),
                        preferred_element_type=jnp.float32)
    r_ref[...] = r + bl_ref[...]
    exT_ref[...] = f_ref[IN:, :]


def _tc_pre(fT, wlT, wrT, bl2):
    nb = (N + BA - 1) // BA
    return pl.pallas_call(
        _tc_pre_body,
        grid=(nb,),
        in_specs=[
            pl.BlockSpec((1044, BA), lambda i: (0, i)),
            pl.BlockSpec((IN, FC), lambda i: (0, 0)),
            pl.BlockSpec((IN, FC), lambda i: (0, 0)),
            pl.BlockSpec((1, FC), lambda i: (0, 0)),
        ],
        out_specs=[
            pl.BlockSpec((BA, FC), lambda i: (i, 0)),
            pl.BlockSpec((BA, FC), lambda i: (i, 0)),
            pl.BlockSpec((EX, BA), lambda i: (0, i)),
        ],
        out_shape=[
            jax.ShapeDtypeStruct((N, FC), jnp.float32),
            jax.ShapeDtypeStruct((N, FC), jnp.float32),
            jax.ShapeDtypeStruct((EX, N), jnp.float32),
        ],
    )(fT, wlT, wrT, bl2)


def _sc_body(z_hbm, src_hbm, dst_hbm, zlc_hbm, agg_hbm, cnt_hbm,
             srcv, dstv, csrc, cdst, gb0, gb1, gb2, zbuf, lc, idxc, acc, accc,
             sg0, sg1, sg2, ss0, ss1, ss2, semz, semlc, semco):
    gbufs = (gb0, gb1, gb2)
    semg = (sg0, sg1, sg2)
    sems = (ss0, ss1, ss2)
    c = lax.axis_index("c")
    s = lax.axis_index("s")
    base_e = pl.multiple_of(s * Et, 8)
    pltpu.sync_copy(src_hbm.at[pl.ds(base_e, Et)], srcv)
    pltpu.sync_copy(dst_hbm.at[pl.ds(base_e, Et)], dstv)

    def _zb(rr, _):
        for j in range(FC // L):
            zbuf[rr, pl.ds(j * L, L)] = jnp.zeros((L,), jnp.float32)
        return 0
    lax.fori_loop(0, ZR, _zb, 0)
    for j in range(CRP // L):
        idxc[0, pl.ds(j * L, L)] = lax.iota(jnp.int32, L) + j * L

    def _wait_copyout():
        pltpu.make_async_copy(acc.at[pl.ds(0, CT)],
                              agg_hbm.at[pl.ds(0, CT)], semco).wait()

        @pl.when(s == 0)
        def _():
            pltpu.make_async_copy(accc.at[pl.ds(0, CR)],
                                  cnt_hbm.at[pl.ds(0, CR)], semco).wait()

    for p in range(P):
        lo = (c * P + p) * C

        if p > 0:
            _wait_copyout()

        for t in range(NZ):
            pltpu.async_copy(
                zbuf, acc.at[pl.ds(pl.multiple_of(s * CT + t * ZR, 8), ZR)], semz)

        @pl.when(s == 0)
        def _():
            pltpu.async_copy(zbuf.at[pl.ds(0, 8)], acc.at[pl.ds(C, 8)], semz)
            for t in range(CRP // ZR):
                pltpu.async_copy(zbuf, accc.at[pl.ds(t * ZR, ZR)], semz)

        for t in range(CRP // ZR):
            pltpu.async_copy(zlc_hbm, lc.at[pl.ds(t * ZR, ZR)], semlc)
        for t in range(CRP // ZR):
            pltpu.make_async_copy(zlc_hbm, lc.at[pl.ds(0, ZR)], semlc).wait()

        lov = jnp.full((L,), lo, jnp.int32)

        def _comp(i, cnt):
            off = pl.multiple_of(i * L, L)
            d = dstv[pl.ds(off, L)]
            sv = srcv[pl.ds(off, L)]
            dl = d - lov
            m = (dl >= jnp.zeros((L,), jnp.int32)) & (dl < jnp.full((L,), C, jnp.int32))
            mi = m.astype(jnp.int32)
            pos = jnp.full((L,), cnt, jnp.int32) + plsc.cumsum(mi) - mi
            plsc.store_scatter(csrc, [pos], sv, mask=m)
            dr = lax.shift_right_logical(dl, 7)
            dc = lax.bitwise_and(dl, jnp.full((L,), FC - 1, jnp.int32))
            plsc.store_scatter(
                cdst,
                [lax.shift_right_logical(pos, 7),
                 lax.bitwise_and(pos, jnp.full((L,), G - 1, jnp.int32))],
                dl, mask=m)
            occ, lastm = plsc.scan_count(dl, m)
            plsc.addupdate_scatter(lc, [dr, dc], occ.astype(jnp.float32),
                                   mask=lastm & m)
            return cnt + jnp.sum(mi)

        cnt = lax.fori_loop(0, NV, _comp, jnp.int32(0))
        ngroups = lax.shift_right_logical(cnt + (G - 1), 7)

        @pl.when(cnt > 0)
        def _():
            g0 = ngroups - 1
            for j in range(G // L):
                idx = jnp.full((L,), g0 * G + j * L, jnp.int32) + lax.iota(jnp.int32, L)
                m = idx >= jnp.full((L,), cnt, jnp.int32)
                plsc.store_scatter(csrc, [idx], jnp.zeros((L,), jnp.int32), mask=m)
                plsc.store_scatter(
                    cdst,
                    [jnp.full((L,), g0, jnp.int32),
                     lax.iota(jnp.int32, L) + j * L],
                    jnp.full((L,), C, jnp.int32), mask=m)

        for t in range(NZ):
            pltpu.make_async_copy(zbuf, acc.at[pl.ds(0, ZR)], semz).wait()

        @pl.when(s == 0)
        def _():
            pltpu.make_async_copy(zbuf.at[pl.ds(0, 8)], acc.at[pl.ds(C, 8)],
                                  semz).wait()
            for t in range(CRP // ZR):
                pltpu.make_async_copy(zbuf, accc.at[pl.ds(0, ZR)], semz).wait()

        plsc.subcore_barrier()

        nsteps = (ngroups + (NBUF - 1)) // NBUF

        def _super(ss_i, _):
            for b in range(NBUF):
                g = ss_i * NBUF + b

                @pl.when(g < ngroups)
                def _():
                    @pl.when(ss_i > 0)
                    def _():
                        pltpu.make_async_copy(gbufs[b], acc.at[pl.ds(0, G)],
                                              sems[b]).wait()
                    goff = pl.multiple_of(g * G, G)
                    pltpu.async_copy(z_hbm.at[csrc.at[pl.ds(goff, G)]],
                                     gbufs[b], semg[b])
            for b in range(NBUF):
                g = ss_i * NBUF + b

                @pl.when(g < ngroups)
                def _():
                    pltpu.make_async_copy(z_hbm.at[pl.ds(0, G)], gbufs[b],
                                          semg[b]).wait()
                    pltpu.async_copy(gbufs[b], acc.at[cdst.at[g]], sems[b],
                                     add=True)
            return 0

        lax.fori_loop(0, nsteps, _super, 0)
        for b in range(NBUF):
            @pl.when(ngroups > b)
            def _():
                pltpu.make_async_copy(gbufs[b], acc.at[pl.ds(0, G)],
                                      sems[b]).wait()

        pltpu.sync_copy(lc, accc.at[idxc.at[0]], add=True)

        plsc.subcore_barrier()

        row0 = s * CT
        pltpu.async_copy(acc.at[pl.ds(pl.multiple_of(row0, 8), CT)],
                         agg_hbm.at[pl.ds(pl.multiple_of(lo + row0, 8), CT)],
                         semco)

        @pl.when(s == 0)
        def _():
            pltpu.async_copy(accc.at[pl.ds(0, CR)],
                             cnt_hbm.at[pl.ds(pl.multiple_of((c * P + p) * CR, 8), CR)],
                             semco)

    _wait_copyout()


_sc_segsum = functools.partial(
    pl.kernel,
    out_type=(
        jax.ShapeDtypeStruct((NPAD, FC), jnp.float32),
        jax.ShapeDtypeStruct((NPAD // FC, FC), jnp.float32),
    ),
    mesh=plsc.VectorSubcoreMesh(core_axis_name="c", subcore_axis_name="s"),
    compiler_params=pltpu.CompilerParams(needs_layout_passes=False),
    scratch_types=[
        pltpu.VMEM((Et,), jnp.int32),
        pltpu.VMEM((Et,), jnp.int32),
        pltpu.VMEM((MAXC,), jnp.int32),
        pltpu.VMEM((MAXG, G), jnp.int32),
        pltpu.VMEM((G, FC), jnp.float32),
        pltpu.VMEM((G, FC), jnp.float32),
        pltpu.VMEM((G, FC), jnp.float32),
        pltpu.VMEM((ZR, FC), jnp.float32),
        pltpu.VMEM((CRP, FC), jnp.float32),
        pltpu.VMEM((8, CRP), jnp.int32),
        pltpu.VMEM_SHARED((C + 8, FC), jnp.float32),
        pltpu.VMEM_SHARED((CRP, FC), jnp.float32),
        pltpu.SemaphoreType.DMA,
        pltpu.SemaphoreType.DMA,
        pltpu.SemaphoreType.DMA,
        pltpu.SemaphoreType.DMA,
        pltpu.SemaphoreType.DMA,
        pltpu.SemaphoreType.DMA,
        pltpu.SemaphoreType.DMA,
        pltpu.SemaphoreType.DMA,
        pltpu.SemaphoreType.DMA,
    ],
)(_sc_body)


def _tc_post_body(agg_ref, cnt_ref, rsel_ref, lsel_ref, r_ref, exT_ref,
                  w1aT_ref, w1bT_ref, b1_ref,
                  gamma_ref, beta_ref, w2T_ref, b2_ref, out_ref):
    cb = cnt_ref[0]
    cr = lax.dot_general(rsel_ref[...], cb, (((1,), (0,)), ((), ())),
                         preferred_element_type=jnp.float32)
    cntcol = jnp.sum(cr * lsel_ref[...], axis=1, keepdims=True)
    mean = agg_ref[...] / jnp.maximum(cntcol, 1.0)
    h = jnp.maximum(mean + r_ref[...], 0.0)
    y = lax.dot_general(h, w1aT_ref[...], (((1,), (0,)), ((), ())),
                        preferred_element_type=jnp.float32)
    y = y + lax.dot_general(exT_ref[...], w1bT_ref[...], (((0,), (0,)), ((), ())),
                            preferred_element_type=jnp.float32)
    y = jnp.maximum(y + b1_ref[...], 0.0)
    scale = gamma_ref[...] * (1.0 / (1.0 + EPS) ** 0.5)
    y = y * scale + beta_ref[...]
    out_ref[...] = lax.dot_general(y, w2T_ref[...], (((1,), (0,)), ((), ())),
                                   preferred_element_type=jnp.float32) + b2_ref[...]


def _tc_post(agg, cnt3d, rsel, lsel, r, exT, w1aT, w1bT, b12, gamma2, beta2,
             w2T, b22):
    nb = (N + BN2 - 1) // BN2
    return pl.pallas_call(
        _tc_post_body,
        grid=(nb,),
        in_specs=[
            pl.BlockSpec((BN2, FC), lambda i: (i, 0)),
            pl.BlockSpec((1, BN2 // FC, FC), lambda i: (i, 0, 0)),
            pl.BlockSpec((BN2, BN2 // FC), lambda i: (0, 0)),
            pl.BlockSpec((BN2, FC), lambda i: (0, 0)),
            pl.BlockSpec((BN2, FC), lambda i: (i, 0)),
            pl.BlockSpec((EX, BN2), lambda i: (0, i)),
            pl.BlockSpec((FC, HID), lambda i: (0, 0)),
            pl.BlockSpec((EX, HID), lambda i: (0, 0)),
            pl.BlockSpec((1, HID), lambda i: (0, 0)),
            pl.BlockSpec((1, HID), lambda i: (0, 0)),
            pl.BlockSpec((1, HID), lambda i: (0, 0)),
            pl.BlockSpec((HID, 3), lambda i: (0, 0)),
            pl.BlockSpec((1, 3), lambda i: (0, 0)),
        ],
        out_specs=pl.BlockSpec((BN2, 3), lambda i: (i, 0)),
        out_shape=jax.ShapeDtypeStruct((N, 3), jnp.float32),
    )(agg, cnt3d, rsel, lsel, r, exT, w1aT, w1bT, b12, gamma2, beta2, w2T, b22)


def kernel(features, edges, edges2, edge_features, Wl, bl, Wr, W1, b1,
           gamma, beta, W2, b2):
    del edges2, edge_features
    src = edges[0]
    dst = edges[1]
    srcp = jnp.concatenate([src, jnp.zeros((EPAD - E,), jnp.int32)])
    dstp = jnp.concatenate([dst, jnp.full((EPAD - E,), NPAD, jnp.int32)])

    z, r, exT = _tc_pre(features.T, Wl.T.astype(jnp.bfloat16),
                        Wr.T.astype(jnp.bfloat16), bl.reshape(1, FC))
    agg, cnt2d = _sc_segsum(z, srcp, dstp, jnp.zeros((ZR, FC), jnp.float32))
    cnt3d = cnt2d.reshape(NPAD // BN2, BN2 // FC, FC)
    nidx = jnp.arange(BN2, dtype=jnp.int32)
    rsel = (nidx[:, None] // FC == jnp.arange(BN2 // FC, dtype=jnp.int32)[None, :]
            ).astype(jnp.float32)
    lsel = (nidx[:, None] % FC == jnp.arange(FC, dtype=jnp.int32)[None, :]
            ).astype(jnp.float32)
    out = _tc_post(agg, cnt3d, rsel, lsel, r, exT, W1[:, :FC].T, W1[:, FC:].T,
                   b1.reshape(1, HID), gamma.reshape(1, HID),
                   beta.reshape(1, HID), W2.T, b2.reshape(1, 3))
    return out
